# Initial kernel scaffold; baseline (speedup 1.0000x reference)
import jax
import jax.numpy as jnp
from jax.experimental import pallas as pl


def kernel(x, pos_table):
    raise NotImplementedError("write your pallas kernel here")


# speedup vs baseline: 2.8186x; 2.8186x over previous
import jax
import jax.numpy as jnp
from jax.experimental import pallas as pl

_TS = 512


def _add_kernel(x_ref, pos_ref, out_ref):
    out_ref[...] = x_ref[...] + pos_ref[...][None, :, :]


def kernel(x, pos_table):
    batch, seq_len, dim = x.shape
    grid = (seq_len // _TS, batch)
    return pl.pallas_call(
        _add_kernel,
        grid=grid,
        in_specs=[
            pl.BlockSpec((1, _TS, dim), lambda s, b: (b, s, 0)),
            pl.BlockSpec((_TS, dim), lambda s, b: (s, 0)),
        ],
        out_specs=pl.BlockSpec((1, _TS, dim), lambda s, b: (b, s, 0)),
        out_shape=jax.ShapeDtypeStruct((batch, seq_len, dim), x.dtype),
    )(x, pos_table[:seq_len])


# speedup vs baseline: 3.1647x; 1.1228x over previous
import jax
import jax.numpy as jnp
from jax.experimental import pallas as pl

_TS = 1024


def _add_kernel(x_ref, pos_ref, out_ref):
    out_ref[...] = x_ref[...] + pos_ref[...][None, :, :]


def kernel(x, pos_table):
    batch, seq_len, dim = x.shape
    grid = (seq_len // _TS, batch)
    return pl.pallas_call(
        _add_kernel,
        grid=grid,
        in_specs=[
            pl.BlockSpec((1, _TS, dim), lambda s, b: (b, s, 0)),
            pl.BlockSpec((_TS, dim), lambda s, b: (s, 0)),
        ],
        out_specs=pl.BlockSpec((1, _TS, dim), lambda s, b: (b, s, 0)),
        out_shape=jax.ShapeDtypeStruct((batch, seq_len, dim), x.dtype),
    )(x, pos_table[:seq_len])


# speedup vs baseline: 3.3067x; 1.0449x over previous
import jax
import jax.numpy as jnp
from jax.experimental import pallas as pl

_TS = 2048


def _add_kernel(x_ref, pos_ref, out_ref):
    out_ref[...] = x_ref[...] + pos_ref[...][None, :, :]


def kernel(x, pos_table):
    batch, seq_len, dim = x.shape
    grid = (seq_len // _TS, batch)
    return pl.pallas_call(
        _add_kernel,
        grid=grid,
        in_specs=[
            pl.BlockSpec((1, _TS, dim), lambda s, b: (b, s, 0)),
            pl.BlockSpec((_TS, dim), lambda s, b: (s, 0)),
        ],
        out_specs=pl.BlockSpec((1, _TS, dim), lambda s, b: (b, s, 0)),
        out_shape=jax.ShapeDtypeStruct((batch, seq_len, dim), x.dtype),
    )(x, pos_table[:seq_len])
